# Initial kernel scaffold; baseline (speedup 1.0000x reference)
import jax
import jax.numpy as jnp
from jax.experimental import pallas as pl


def kernel(labels, masks, vertex_pred, extents, poses, meta_data):
    raise NotImplementedError("write your pallas kernel here")


# speedup vs baseline: 1.0282x; 1.0282x over previous
import numpy as np
import jax
import jax.numpy as jnp
from jax import lax
from jax.experimental import pallas as pl
from jax.experimental.pallas import tpu as pltpu
from jax.experimental.pallas import tpu_sc as plsc

B, H, W, NCLS = 2, 480, 640, 22
HW = H * W
NPIX = B * HW
NC, NS = 2, 16
NW = NC * NS
PPW = NPIX // NW
ROWS_W = PPW // W
NCH = 10
CPP = PPW // NCH
CROWS = CPP // W
JROW = W // 16
GJ = CPP // 16
PIXR = CPP // 128
IDXR = 3 * PIXR
PROWS_W = PPW // 128
NBIN = B * (NCLS - 1)
ACCW = NBIN * 8
VOTESTRIDE = 48
EPS = 1e-6
F32 = jnp.float32


def _rsqrt(n2):
    i = lax.bitcast_convert_type(n2, jnp.int32)
    i = 0x5F3759DF - (i >> 1)
    r = lax.bitcast_convert_type(i, F32)
    for _ in range(3):
        r = r * (1.5 - 0.5 * n2 * r * r)
    return r


def _p1_body(lab_hbm, msk_hbm, vp_hbm,
             part_hbm, dxm_hbm, dym_hbm, labe_hbm,
             lab_v, msk_v, idx_v, g_v, acc_v, sem):
    cid = lax.axis_index("c")
    sid = lax.axis_index("s")
    wid = cid * NS + sid
    b = wid // NS
    wm = wid - b * NS
    base = wid * PPW
    qbase = wm * PPW
    row0 = wm * ROWS_W
    lanes = lax.iota(jnp.int32, 16)

    pltpu.sync_copy(lab_hbm.at[pl.ds(base, PPW)], lab_v)
    pltpu.sync_copy(msk_hbm.at[pl.ds(base, PPW)], msk_v)

    def _zero(i, _):
        acc_v[pl.ds(i * 16, 16)] = jnp.zeros((16,), F32)
        return 0
    lax.fori_loop(0, (16 * ACCW) // 16, _zero, 0)

    def _chunk(c, _):
        pc = c * CPP

        def _bld(j, carry):
            off = pc + j * 16
            l0 = lab_v[pl.ds(off, 16)]
            mk = msk_v[pl.ds(off, 16)]
            le = jnp.where((mk > 0) & (l0 > 0), l0, 0)
            lab_v[pl.ds(off, 16)] = le
            q = qbase + off + lanes
            i0 = b * (66 * HW) + 3 * le * HW + q
            jr = j >> 3
            jcol = (j & 7) * 16
            idx_v[jr, pl.ds(jcol, 16)] = i0
            idx_v[jr + PIXR, pl.ds(jcol, 16)] = i0 + HW
            idx_v[jr + 2 * PIXR, pl.ds(jcol, 16)] = i0 + 2 * HW
            return carry
        lax.fori_loop(0, GJ, _bld, 0)

        def _batch(g, carry):
            def _fire(i, cc):
                r = g * PIXR + i
                pltpu.async_copy(vp_hbm.at[idx_v.at[r]],
                                 g_v.at[pl.ds(r * 128, 128)], sem)
                return cc
            lax.fori_loop(0, PIXR, _fire, 0)
            pltpu.make_async_copy(dxm_hbm.at[pl.ds(0, CPP)],
                                  g_v.at[pl.ds(g * CPP, CPP)], sem).wait()
            return carry
        lax.fori_loop(0, 3, _batch, 0)

        def _row(rr, carry):
            yf = (row0 + c * CROWS + rr).astype(F32)

            def _grp(jj, cc):
                j = rr * JROW + jj
                jr = j >> 3
                jcol = (j & 7) * 16
                off = pc + j * 16
                le = lab_v[pl.ds(off, 16)]
                gx = g_v[pl.ds(j * 16, 16)]
                gy = g_v[pl.ds(CPP + j * 16, 16)]
                gz = g_v[pl.ds(2 * CPP + j * 16, 16)]
                valid = le > 0
                n2 = jnp.maximum(gx * gx + gy * gy, 1e-12)
                r = _rsqrt(n2)
                s = r / (1.0 + EPS * r)
                dx = gx * s
                dy = gy * s
                zero = jnp.zeros((16,), F32)
                dxm = jnp.where(valid, dx, zero)
                dym = jnp.where(valid, dy, zero)
                g_v[pl.ds(j * 16, 16)] = dxm
                g_v[pl.ds(CPP + j * 16, 16)] = dym
                xf = (jj * 16 + lanes).astype(F32)
                t1 = 1.0 - dx * dx
                t2 = -dx * dy
                t3 = 1.0 - dy * dy
                s1 = t1 * xf + t2 * yf
                s2 = t2 * xf + t3 * yf
                binv = b * (NCLS - 1) + jnp.maximum(le - 1, 0)
                ab = lanes * ACCW + binv * 8
                one = jnp.ones((16,), F32)
                plsc.addupdate_scatter(acc_v, [ab], one, mask=valid)
                plsc.addupdate_scatter(acc_v, [ab + 1], t1, mask=valid)
                plsc.addupdate_scatter(acc_v, [ab + 2], t2, mask=valid)
                plsc.addupdate_scatter(acc_v, [ab + 3], t3, mask=valid)
                plsc.addupdate_scatter(acc_v, [ab + 4], s1, mask=valid)
                plsc.addupdate_scatter(acc_v, [ab + 5], s2, mask=valid)
                plsc.addupdate_scatter(acc_v, [ab + 6], gz, mask=valid)
                return cc
            lax.fori_loop(0, JROW, _grp, 0)
            return carry
        lax.fori_loop(0, CROWS, _row, 0)

        pltpu.sync_copy(g_v.at[pl.ds(0, CPP)],
                        dxm_hbm.at[pl.ds(base + pc, CPP)])
        pltpu.sync_copy(g_v.at[pl.ds(CPP, CPP)],
                        dym_hbm.at[pl.ds(base + pc, CPP)])
        return 0
    lax.fori_loop(0, NCH, _chunk, 0)

    pltpu.sync_copy(lab_v, labe_hbm.at[pl.ds(base, PPW)])

    def _fold(t, _):
        v = acc_v[pl.ds(t * 16, 16)]
        for l in range(1, 16):
            v = v + acc_v[pl.ds(l * ACCW + t * 16, 16)]
        acc_v[pl.ds(t * 16, 16)] = v
        return 0
    lax.fori_loop(0, ACCW // 16, _fold, 0)
    pltpu.sync_copy(acc_v.at[pl.ds(0, ACCW)],
                    part_hbm.at[pl.ds(wid * ACCW, ACCW)])


def _p2_body(labe_hbm, dxm_hbm, dym_hbm, part_hbm,
             votes_hbm, sums_hbm,
             lab_v, dx_v, dy_v, part_v, cx_v, cy_v, vacc_v):
    cid = lax.axis_index("c")
    sid = lax.axis_index("s")
    wid = cid * NS + sid
    b = wid // NS
    wm = wid - b * NS
    base = wid * PPW
    row0 = wm * ROWS_W
    lanes = lax.iota(jnp.int32, 16)

    pltpu.sync_copy(part_hbm, part_v)

    def _red(t, _):
        v = part_v[pl.ds(t * 16, 16)]
        for wk in range(1, NW):
            v = v + part_v[pl.ds(wk * ACCW + t * 16, 16)]
        part_v[pl.ds(t * 16, 16)] = v
        return 0
    lax.fori_loop(0, ACCW // 16, _red, 0)

    @pl.when(wid == 0)
    def _():
        pltpu.sync_copy(part_v.at[pl.ds(0, ACCW)], sums_hbm)

    def _slv(t, _):
        binv = t * 16 + lanes
        bi8 = binv * 8
        a11 = plsc.load_gather(part_v, [bi8 + 1])
        a12 = plsc.load_gather(part_v, [bi8 + 2])
        a22 = plsc.load_gather(part_v, [bi8 + 3])
        b1 = plsc.load_gather(part_v, [bi8 + 4])
        b2 = plsc.load_gather(part_v, [bi8 + 5])
        det = a11 * a22 - a12 * a12
        det = jnp.where(jnp.abs(det) < EPS, jnp.full((16,), EPS, F32), det)
        cx_v[pl.ds(t * 16, 16)] = (a22 * b1 - a12 * b2) / det
        cy_v[pl.ds(t * 16, 16)] = (a11 * b2 - a12 * b1) / det
        return 0
    lax.fori_loop(0, 3, _slv, 0)

    pltpu.sync_copy(labe_hbm.at[pl.ds(base, PPW)], lab_v)
    pltpu.sync_copy(dxm_hbm.at[pl.ds(base, PPW)], dx_v)
    pltpu.sync_copy(dym_hbm.at[pl.ds(base, PPW)], dy_v)

    def _zero(i, _):
        vacc_v[pl.ds(i * 16, 16)] = jnp.zeros((16,), F32)
        return 0
    lax.fori_loop(0, (16 * VOTESTRIDE) // 16, _zero, 0)

    def _row(rr, carry):
        yf = (row0 + rr).astype(F32)

        def _grp(jj, cc):
            j = rr * JROW + jj
            le = lab_v[pl.ds(j * 16, 16)]
            dxm = dx_v[pl.ds(j * 16, 16)]
            dym = dy_v[pl.ds(j * 16, 16)]
            binv = b * (NCLS - 1) + jnp.maximum(le - 1, 0)
            cx = plsc.load_gather(cx_v, [binv])
            cy = plsc.load_gather(cy_v, [binv])
            xf = (jj * 16 + lanes).astype(F32)
            ux = cx - xf
            uy = cy - yf
            u2 = jnp.maximum(ux * ux + uy * uy, 1e-12)
            r2 = _rsqrt(u2)
            s2 = r2 / (1.0 + EPS * r2)
            dot = (ux * dxm + uy * dym) * s2
            vm = dot > 0.9
            one = jnp.ones((16,), F32)
            plsc.addupdate_scatter(vacc_v, [lanes * VOTESTRIDE + binv], one,
                                   mask=vm)
            return cc
        lax.fori_loop(0, JROW, _grp, 0)
        return carry
    lax.fori_loop(0, ROWS_W, _row, 0)

    def _foldv(t, _):
        v = vacc_v[pl.ds(t * 16, 16)]
        for l in range(1, 16):
            v = v + vacc_v[pl.ds(l * VOTESTRIDE + t * 16, 16)]
        vacc_v[pl.ds(t * 16, 16)] = v
        return 0
    lax.fori_loop(0, VOTESTRIDE // 16, _foldv, 0)
    pltpu.sync_copy(vacc_v.at[pl.ds(0, VOTESTRIDE)],
                    votes_hbm.at[pl.ds(wid * VOTESTRIDE, VOTESTRIDE)])


def _fin_body(sums_ref, votes_ref, ext_ref, poses_ref, meta_ref,
              box_ref, pose_ref):
    sums = sums_ref[...]
    votes = jnp.sum(votes_ref[...], axis=0)[:NBIN]
    cnt = sums[:, 0]
    a11 = sums[:, 1]
    a12 = sums[:, 2]
    a22 = sums[:, 3]
    b1 = sums[:, 4]
    b2 = sums[:, 5]
    sz = sums[:, 6]
    det = a11 * a22 - a12 * a12
    det = jnp.where(jnp.abs(det) < EPS, EPS, det)
    cx = (a22 * b1 - a12 * b2) / det
    cy = (a11 * b2 - a12 * b1) / det

    bsel = lax.broadcasted_iota(jnp.int32, (NBIN,), 0) >= (NCLS - 1)
    fx = jnp.where(bsel, meta_ref[1, 0], meta_ref[0, 0]) + EPS
    fy = jnp.where(bsel, meta_ref[1, 4], meta_ref[0, 4]) + EPS
    px0 = jnp.where(bsel, meta_ref[1, 2], meta_ref[0, 2])
    py0 = jnp.where(bsel, meta_ref[1, 5], meta_ref[0, 5])

    frac = votes / jnp.maximum(cnt, 1.0)
    depth = jnp.exp(sz / jnp.maximum(cnt, 1.0))
    tx = depth * (cx - px0) / fx
    ty = depth * (cy - py0) / fy

    e = ext_ref[...]
    ext2 = jnp.sum(e * e, axis=1)
    e21 = ext2[1:NCLS]
    ext2sel = jnp.concatenate([e21, e21])
    extv = jnp.sqrt(ext2sel + EPS)
    bw = 0.5 * fx * extv / (depth + EPS)
    bh = 0.5 * fy * extv / (depth + EPS)
    validv = ((cnt > 500.0) & (votes >= 100.0)).astype(F32)
    score = frac * validv

    rowi = lax.broadcasted_iota(jnp.int32, (NBIN,), 0)
    clsv = (rowi + 1 - (NCLS - 1) * bsel.astype(jnp.int32)).astype(F32)
    box_ref[0, :] = bsel.astype(F32)
    box_ref[1, :] = clsv
    box_ref[2, :] = cx - bw
    box_ref[3, :] = cy - bh
    box_ref[4, :] = cx + bw
    box_ref[5, :] = cy + bh
    box_ref[6, :] = score

    for k in range(4):
        pose_ref[k, :] = jnp.where(bsel, poses_ref[1, 6 + k],
                                   poses_ref[0, 6 + k])
    pose_ref[4, :] = tx
    pose_ref[5, :] = ty
    pose_ref[6, :] = depth


_mesh = plsc.VectorSubcoreMesh(core_axis_name="c", subcore_axis_name="s")

_pass1 = pl.kernel(
    _p1_body,
    out_type=[
        jax.ShapeDtypeStruct((NW * ACCW,), F32),
        jax.ShapeDtypeStruct((NPIX,), F32),
        jax.ShapeDtypeStruct((NPIX,), F32),
        jax.ShapeDtypeStruct((NPIX,), jnp.int32),
    ],
    mesh=_mesh,
    compiler_params=pltpu.CompilerParams(needs_layout_passes=False),
    scratch_types=[
        pltpu.VMEM((PPW,), jnp.int32),
        pltpu.VMEM((PPW,), jnp.int32),
        pltpu.VMEM((IDXR, 128), jnp.int32),
        pltpu.VMEM((3 * CPP,), F32),
        pltpu.VMEM((16 * ACCW,), F32),
        pltpu.SemaphoreType.DMA,
    ],
)

_pass2 = pl.kernel(
    _p2_body,
    out_type=[
        jax.ShapeDtypeStruct((NW * VOTESTRIDE,), F32),
        jax.ShapeDtypeStruct((ACCW,), F32),
    ],
    mesh=_mesh,
    compiler_params=pltpu.CompilerParams(needs_layout_passes=False),
    scratch_types=[
        pltpu.VMEM((PPW,), jnp.int32),
        pltpu.VMEM((PPW,), F32),
        pltpu.VMEM((PPW,), F32),
        pltpu.VMEM((NW * ACCW,), F32),
        pltpu.VMEM((VOTESTRIDE,), F32),
        pltpu.VMEM((VOTESTRIDE,), F32),
        pltpu.VMEM((16 * VOTESTRIDE,), F32),
    ],
)

_final = pl.pallas_call(
    _fin_body,
    out_shape=[
        jax.ShapeDtypeStruct((7, NBIN), F32),
        jax.ShapeDtypeStruct((7, NBIN), F32),
    ],
)


def kernel(labels, masks, vertex_pred, extents, poses, meta_data):
    labf = labels.reshape(NPIX)
    mskf = masks.reshape(NPIX)
    vpf = vertex_pred.reshape(-1)
    part, dxm, dym, labe = _pass1(labf, mskf, vpf)
    votes, sums = _pass2(labe, dxm, dym, part)
    boxt, poset = _final(sums.reshape(NBIN, 8), votes.reshape(NW, VOTESTRIDE),
                         extents, poses, meta_data)
    top_box = boxt.T
    top_pose = poset.T
    top_target = jnp.zeros((NBIN, 4 * NCLS), F32)
    top_weight = jnp.zeros((NBIN, 4 * NCLS), F32)
    top_domain = jnp.repeat(jnp.arange(B, dtype=F32), NCLS - 1)
    return top_box, top_pose, top_target, top_weight, top_domain


# speedup vs baseline: 2.0370x; 1.9811x over previous
import numpy as np
import jax
import jax.numpy as jnp
from jax import lax
from jax.experimental import pallas as pl
from jax.experimental.pallas import tpu as pltpu
from jax.experimental.pallas import tpu_sc as plsc

B, H, W, NCLS = 2, 480, 640, 22
HW = H * W
NPIX = B * HW
RH = B * H
NC, NS = 2, 16
NW = NC * NS
NUNIT = RH // 8
JROW = W // 16
NBIN = B * (NCLS - 1)
ACCW = NBIN * 8
VOTESTRIDE = 48
BH = 16
EPS = 1e-6
F32 = jnp.float32


def _rsqrt(n2):
    i = lax.bitcast_convert_type(n2, jnp.int32)
    i = 0x5F3759DF - (i >> 1)
    r = lax.bitcast_convert_type(i, F32)
    for _ in range(3):
        r = r * (1.5 - 0.5 * n2 * r * r)
    return r


def _sel_body(lab_ref, msk_ref, vp_ref, labe_ref, dxm_ref, dym_ref, vzm_ref):
    lab = lab_ref[...]
    msk = msk_ref[...]
    le = jnp.where((msk > 0) & (lab > 0), lab, 0)
    vx = vp_ref[0, 0]
    vy = vp_ref[0, 1]
    vz = vp_ref[0, 2]
    for c in range(1, NCLS):
        mc = le == c
        vx = jnp.where(mc, vp_ref[0, 3 * c], vx)
        vy = jnp.where(mc, vp_ref[0, 3 * c + 1], vy)
        vz = jnp.where(mc, vp_ref[0, 3 * c + 2], vz)
    valid = le > 0
    n2 = jnp.maximum(vx * vx + vy * vy, 1e-12)
    r = lax.rsqrt(n2)
    s = r / (1.0 + EPS * r)
    zero = jnp.zeros_like(vx)
    labe_ref[...] = le
    dxm_ref[...] = jnp.where(valid, vx * s, zero)
    dym_ref[...] = jnp.where(valid, vy * s, zero)
    vzm_ref[...] = jnp.where(valid, vz, zero)


_plane_spec = pl.BlockSpec((BH, W), lambda g: (g, 0))

_select = pl.pallas_call(
    _sel_body,
    grid=(RH // BH,),
    in_specs=[
        _plane_spec,
        _plane_spec,
        pl.BlockSpec((1, 3 * NCLS, BH, W),
                     lambda g: (g // (H // BH), 0, g % (H // BH), 0)),
    ],
    out_specs=[_plane_spec, _plane_spec, _plane_spec, _plane_spec],
    out_shape=[
        jax.ShapeDtypeStruct((RH, W), jnp.int32),
        jax.ShapeDtypeStruct((RH, W), F32),
        jax.ShapeDtypeStruct((RH, W), F32),
        jax.ShapeDtypeStruct((RH, W), F32),
    ],
)


def _mom_body(labe_hbm, dxm_hbm, dym_hbm, vzm_hbm, part_hbm,
              le_v, dx_v, dy_v, vz_v, acc_v):
    cid = lax.axis_index("c")
    sid = lax.axis_index("s")
    wid = cid * NS + sid
    b = wid // NS
    u0 = (wid * NUNIT) // NW
    u1 = ((wid + 1) * NUNIT) // NW
    lanes = lax.iota(jnp.int32, 16)

    def _zero(i, _):
        acc_v[pl.ds(i * 16, 16)] = jnp.zeros((16,), F32)
        return 0
    lax.fori_loop(0, (16 * ACCW) // 16, _zero, 0)

    def _unit(u, _):
        r0 = u * 8
        pltpu.sync_copy(labe_hbm.at[pl.ds(r0, 8)], le_v)
        pltpu.sync_copy(dxm_hbm.at[pl.ds(r0, 8)], dx_v)
        pltpu.sync_copy(dym_hbm.at[pl.ds(r0, 8)], dy_v)
        pltpu.sync_copy(vzm_hbm.at[pl.ds(r0, 8)], vz_v)

        def _row(rr, carry):
            yf = (r0 + rr - b * H).astype(F32)

            def _grp(jj, cc):
                cs = jj * 16
                le = le_v[rr, pl.ds(cs, 16)]
                dx = dx_v[rr, pl.ds(cs, 16)]
                dy = dy_v[rr, pl.ds(cs, 16)]
                vz = vz_v[rr, pl.ds(cs, 16)]
                valid = le > 0
                xf = (cs + lanes).astype(F32)
                t1 = 1.0 - dx * dx
                t2 = -dx * dy
                t3 = 1.0 - dy * dy
                s1 = t1 * xf + t2 * yf
                s2 = t2 * xf + t3 * yf
                binv = b * (NCLS - 1) + jnp.maximum(le - 1, 0)
                ab = lanes * ACCW + binv * 8
                one = jnp.ones((16,), F32)
                plsc.addupdate_scatter(acc_v, [ab], one, mask=valid)
                plsc.addupdate_scatter(acc_v, [ab + 1], t1, mask=valid)
                plsc.addupdate_scatter(acc_v, [ab + 2], t2, mask=valid)
                plsc.addupdate_scatter(acc_v, [ab + 3], t3, mask=valid)
                plsc.addupdate_scatter(acc_v, [ab + 4], s1, mask=valid)
                plsc.addupdate_scatter(acc_v, [ab + 5], s2, mask=valid)
                plsc.addupdate_scatter(acc_v, [ab + 6], vz, mask=valid)
                return cc
            lax.fori_loop(0, JROW, _grp, 0)
            return carry
        lax.fori_loop(0, 8, _row, 0)
        return 0
    lax.fori_loop(u0, u1, _unit, 0)

    def _fold(t, _):
        v = acc_v[pl.ds(t * 16, 16)]
        for l in range(1, 16):
            v = v + acc_v[pl.ds(l * ACCW + t * 16, 16)]
        acc_v[pl.ds(t * 16, 16)] = v
        return 0
    lax.fori_loop(0, ACCW // 16, _fold, 0)
    pltpu.sync_copy(acc_v.at[pl.ds(0, ACCW)],
                    part_hbm.at[pl.ds(wid * ACCW, ACCW)])


def _vote_body(labe_hbm, dxm_hbm, dym_hbm, part_hbm,
               votes_hbm, sums_hbm,
               le_v, dx_v, dy_v, part_v, cx_v, cy_v, vacc_v):
    cid = lax.axis_index("c")
    sid = lax.axis_index("s")
    wid = cid * NS + sid
    b = wid // NS
    u0 = (wid * NUNIT) // NW
    u1 = ((wid + 1) * NUNIT) // NW
    lanes = lax.iota(jnp.int32, 16)

    pltpu.sync_copy(part_hbm, part_v)

    def _red(t, _):
        v = part_v[pl.ds(t * 16, 16)]
        for wk in range(1, NW):
            v = v + part_v[pl.ds(wk * ACCW + t * 16, 16)]
        part_v[pl.ds(t * 16, 16)] = v
        return 0
    lax.fori_loop(0, ACCW // 16, _red, 0)

    @pl.when(wid == 0)
    def _():
        pltpu.sync_copy(part_v.at[pl.ds(0, ACCW)], sums_hbm)

    def _slv(t, _):
        binv = t * 16 + lanes
        bi8 = binv * 8
        a11 = plsc.load_gather(part_v, [bi8 + 1])
        a12 = plsc.load_gather(part_v, [bi8 + 2])
        a22 = plsc.load_gather(part_v, [bi8 + 3])
        b1 = plsc.load_gather(part_v, [bi8 + 4])
        b2 = plsc.load_gather(part_v, [bi8 + 5])
        det = a11 * a22 - a12 * a12
        det = jnp.where(jnp.abs(det) < EPS, jnp.full((16,), EPS, F32), det)
        cx_v[pl.ds(t * 16, 16)] = (a22 * b1 - a12 * b2) / det
        cy_v[pl.ds(t * 16, 16)] = (a11 * b2 - a12 * b1) / det
        return 0
    lax.fori_loop(0, 3, _slv, 0)

    def _zero(i, _):
        vacc_v[pl.ds(i * 16, 16)] = jnp.zeros((16,), F32)
        return 0
    lax.fori_loop(0, (16 * VOTESTRIDE) // 16, _zero, 0)

    def _unit(u, _):
        r0 = u * 8
        pltpu.sync_copy(labe_hbm.at[pl.ds(r0, 8)], le_v)
        pltpu.sync_copy(dxm_hbm.at[pl.ds(r0, 8)], dx_v)
        pltpu.sync_copy(dym_hbm.at[pl.ds(r0, 8)], dy_v)

        def _row(rr, carry):
            yf = (r0 + rr - b * H).astype(F32)

            def _grp(jj, cc):
                cs = jj * 16
                le = le_v[rr, pl.ds(cs, 16)]
                dxm = dx_v[rr, pl.ds(cs, 16)]
                dym = dy_v[rr, pl.ds(cs, 16)]
                binv = b * (NCLS - 1) + jnp.maximum(le - 1, 0)
                cx = plsc.load_gather(cx_v, [binv])
                cy = plsc.load_gather(cy_v, [binv])
                xf = (cs + lanes).astype(F32)
                ux = cx - xf
                uy = cy - yf
                u2 = jnp.maximum(ux * ux + uy * uy, 1e-12)
                r2 = _rsqrt(u2)
                s2 = r2 / (1.0 + EPS * r2)
                dot = (ux * dxm + uy * dym) * s2
                vm = dot > 0.9
                one = jnp.ones((16,), F32)
                plsc.addupdate_scatter(vacc_v, [lanes * VOTESTRIDE + binv],
                                       one, mask=vm)
                return cc
            lax.fori_loop(0, JROW, _grp, 0)
            return carry
        lax.fori_loop(0, 8, _row, 0)
        return 0
    lax.fori_loop(u0, u1, _unit, 0)

    def _foldv(t, _):
        v = vacc_v[pl.ds(t * 16, 16)]
        for l in range(1, 16):
            v = v + vacc_v[pl.ds(l * VOTESTRIDE + t * 16, 16)]
        vacc_v[pl.ds(t * 16, 16)] = v
        return 0
    lax.fori_loop(0, VOTESTRIDE // 16, _foldv, 0)
    pltpu.sync_copy(vacc_v.at[pl.ds(0, VOTESTRIDE)],
                    votes_hbm.at[pl.ds(wid * VOTESTRIDE, VOTESTRIDE)])


def _fin_body(sums_ref, votes_ref, ext_ref, poses_ref, meta_ref,
              box_ref, pose_ref):
    sums = sums_ref[...]
    votes = jnp.sum(votes_ref[...], axis=0)[:NBIN]
    cnt = sums[:, 0]
    a11 = sums[:, 1]
    a12 = sums[:, 2]
    a22 = sums[:, 3]
    b1 = sums[:, 4]
    b2 = sums[:, 5]
    sz = sums[:, 6]
    det = a11 * a22 - a12 * a12
    det = jnp.where(jnp.abs(det) < EPS, EPS, det)
    cx = (a22 * b1 - a12 * b2) / det
    cy = (a11 * b2 - a12 * b1) / det

    bsel = lax.broadcasted_iota(jnp.int32, (NBIN,), 0) >= (NCLS - 1)
    fx = jnp.where(bsel, meta_ref[1, 0], meta_ref[0, 0]) + EPS
    fy = jnp.where(bsel, meta_ref[1, 4], meta_ref[0, 4]) + EPS
    px0 = jnp.where(bsel, meta_ref[1, 2], meta_ref[0, 2])
    py0 = jnp.where(bsel, meta_ref[1, 5], meta_ref[0, 5])

    frac = votes / jnp.maximum(cnt, 1.0)
    depth = jnp.exp(sz / jnp.maximum(cnt, 1.0))
    tx = depth * (cx - px0) / fx
    ty = depth * (cy - py0) / fy

    e = ext_ref[...]
    ext2 = jnp.sum(e * e, axis=1)
    e21 = ext2[1:NCLS]
    ext2sel = jnp.concatenate([e21, e21])
    extv = jnp.sqrt(ext2sel + EPS)
    bw = 0.5 * fx * extv / (depth + EPS)
    bh = 0.5 * fy * extv / (depth + EPS)
    validv = ((cnt > 500.0) & (votes >= 100.0)).astype(F32)
    score = frac * validv

    rowi = lax.broadcasted_iota(jnp.int32, (NBIN,), 0)
    clsv = (rowi + 1 - (NCLS - 1) * bsel.astype(jnp.int32)).astype(F32)
    box_ref[0, :] = bsel.astype(F32)
    box_ref[1, :] = clsv
    box_ref[2, :] = cx - bw
    box_ref[3, :] = cy - bh
    box_ref[4, :] = cx + bw
    box_ref[5, :] = cy + bh
    box_ref[6, :] = score

    for k in range(4):
        pose_ref[k, :] = jnp.where(bsel, poses_ref[1, 6 + k],
                                   poses_ref[0, 6 + k])
    pose_ref[4, :] = tx
    pose_ref[5, :] = ty
    pose_ref[6, :] = depth


_mesh = plsc.VectorSubcoreMesh(core_axis_name="c", subcore_axis_name="s")

_moments = pl.kernel(
    _mom_body,
    out_type=[jax.ShapeDtypeStruct((NW * ACCW,), F32)],
    mesh=_mesh,
    compiler_params=pltpu.CompilerParams(needs_layout_passes=False),
    scratch_types=[
        pltpu.VMEM((8, W), jnp.int32),
        pltpu.VMEM((8, W), F32),
        pltpu.VMEM((8, W), F32),
        pltpu.VMEM((8, W), F32),
        pltpu.VMEM((16 * ACCW,), F32),
    ],
)

_votes = pl.kernel(
    _vote_body,
    out_type=[
        jax.ShapeDtypeStruct((NW * VOTESTRIDE,), F32),
        jax.ShapeDtypeStruct((ACCW,), F32),
    ],
    mesh=_mesh,
    compiler_params=pltpu.CompilerParams(needs_layout_passes=False),
    scratch_types=[
        pltpu.VMEM((8, W), jnp.int32),
        pltpu.VMEM((8, W), F32),
        pltpu.VMEM((8, W), F32),
        pltpu.VMEM((NW * ACCW,), F32),
        pltpu.VMEM((VOTESTRIDE,), F32),
        pltpu.VMEM((VOTESTRIDE,), F32),
        pltpu.VMEM((16 * VOTESTRIDE,), F32),
    ],
)

_final = pl.pallas_call(
    _fin_body,
    out_shape=[
        jax.ShapeDtypeStruct((7, NBIN), F32),
        jax.ShapeDtypeStruct((7, NBIN), F32),
    ],
)


def kernel(labels, masks, vertex_pred, extents, poses, meta_data):
    lab2 = labels.reshape(RH, W)
    msk2 = masks.reshape(RH, W)
    labe, dxm, dym, vzm = _select(lab2, msk2, vertex_pred)
    part, = _moments(labe, dxm, dym, vzm)
    votes, sums = _votes(labe, dxm, dym, part)
    boxt, poset = _final(sums.reshape(NBIN, 8), votes.reshape(NW, VOTESTRIDE),
                         extents, poses, meta_data)
    top_box = boxt.T
    top_pose = poset.T
    top_target = jnp.zeros((NBIN, 4 * NCLS), F32)
    top_weight = jnp.zeros((NBIN, 4 * NCLS), F32)
    top_domain = jnp.repeat(jnp.arange(B, dtype=F32), NCLS - 1)
    return top_box, top_pose, top_target, top_weight, top_domain


# speedup vs baseline: 2.1153x; 1.0384x over previous
import numpy as np
import jax
import jax.numpy as jnp
from jax import lax
from jax.experimental import pallas as pl
from jax.experimental.pallas import tpu as pltpu
from jax.experimental.pallas import tpu_sc as plsc

B, H, W, NCLS = 2, 480, 640, 22
HW = H * W
NPIX = B * HW
RH = B * H
NC, NS = 2, 16
NW = NC * NS
NUNIT = RH // 8
JROW = W // 16
NBIN = B * (NCLS - 1)
ACCW = NBIN * 8
VOTESTRIDE = 48
BH = 16
EPS = 1e-6
F32 = jnp.float32


def _rsqrt(n2):
    i = lax.bitcast_convert_type(n2, jnp.int32)
    i = 0x5F3759DF - (i >> 1)
    r = lax.bitcast_convert_type(i, F32)
    for _ in range(3):
        r = r * (1.5 - 0.5 * n2 * r * r)
    return r


def _sel_body(lab_ref, msk_ref, vp_ref, labe_ref, dxm_ref, dym_ref, vzm_ref):
    lab = lab_ref[...]
    msk = msk_ref[...]
    le = jnp.where((msk > 0) & (lab > 0), lab, 0)
    vx = vp_ref[0, 0]
    vy = vp_ref[0, 1]
    vz = vp_ref[0, 2]
    for c in range(1, NCLS):
        mc = le == c
        vx = jnp.where(mc, vp_ref[0, 3 * c], vx)
        vy = jnp.where(mc, vp_ref[0, 3 * c + 1], vy)
        vz = jnp.where(mc, vp_ref[0, 3 * c + 2], vz)
    valid = le > 0
    n2 = jnp.maximum(vx * vx + vy * vy, 1e-12)
    r = lax.rsqrt(n2)
    s = r / (1.0 + EPS * r)
    zero = jnp.zeros_like(vx)
    labe_ref[...] = le
    dxm_ref[...] = jnp.where(valid, vx * s, zero)
    dym_ref[...] = jnp.where(valid, vy * s, zero)
    vzm_ref[...] = jnp.where(valid, vz, zero)


_plane_spec = pl.BlockSpec((BH, W), lambda g: (g, 0))

_select = pl.pallas_call(
    _sel_body,
    grid=(RH // BH,),
    in_specs=[
        _plane_spec,
        _plane_spec,
        pl.BlockSpec((1, 3 * NCLS, BH, W),
                     lambda g: (g // (H // BH), 0, g % (H // BH), 0)),
    ],
    out_specs=[_plane_spec, _plane_spec, _plane_spec, _plane_spec],
    out_shape=[
        jax.ShapeDtypeStruct((RH, W), jnp.int32),
        jax.ShapeDtypeStruct((RH, W), F32),
        jax.ShapeDtypeStruct((RH, W), F32),
        jax.ShapeDtypeStruct((RH, W), F32),
    ],
)


def _mom_body(labe_hbm, dxm_hbm, dym_hbm, vzm_hbm, part_hbm,
              le_v, dx_v, dy_v, vz_v, acc_v):
    cid = lax.axis_index("c")
    sid = lax.axis_index("s")
    wid = cid * NS + sid
    b = wid // NS
    u0 = (wid * NUNIT) // NW
    u1 = ((wid + 1) * NUNIT) // NW
    lanes = lax.iota(jnp.int32, 16)

    def _zero(i, _):
        acc_v[pl.ds(i * 16, 16)] = jnp.zeros((16,), F32)
        return 0
    lax.fori_loop(0, (16 * ACCW) // 16, _zero, 0)

    def _unit(u, _):
        r0 = u * 8
        pltpu.sync_copy(labe_hbm.at[pl.ds(r0, 8)], le_v)
        pltpu.sync_copy(dxm_hbm.at[pl.ds(r0, 8)], dx_v)
        pltpu.sync_copy(dym_hbm.at[pl.ds(r0, 8)], dy_v)
        pltpu.sync_copy(vzm_hbm.at[pl.ds(r0, 8)], vz_v)

        def _row(rr, carry):
            yf = (r0 + rr - b * H).astype(F32)

            def _grp(j4, cc):
                for k in range(4):
                    cs = j4 * 64 + k * 16
                    le = le_v[rr, pl.ds(cs, 16)]
                    dx = dx_v[rr, pl.ds(cs, 16)]
                    dy = dy_v[rr, pl.ds(cs, 16)]
                    vz = vz_v[rr, pl.ds(cs, 16)]
                    valid = le > 0
                    xf = (cs + lanes).astype(F32)
                    t1 = 1.0 - dx * dx
                    t2 = -dx * dy
                    t3 = 1.0 - dy * dy
                    s1 = t1 * xf + t2 * yf
                    s2 = t2 * xf + t3 * yf
                    binv = b * (NCLS - 1) + jnp.maximum(le - 1, 0)
                    ab = lanes * ACCW + binv * 8
                    one = jnp.ones((16,), F32)
                    plsc.addupdate_scatter(acc_v, [ab], one, mask=valid)
                    plsc.addupdate_scatter(acc_v, [ab + 1], t1, mask=valid)
                    plsc.addupdate_scatter(acc_v, [ab + 2], t2, mask=valid)
                    plsc.addupdate_scatter(acc_v, [ab + 3], t3, mask=valid)
                    plsc.addupdate_scatter(acc_v, [ab + 4], s1, mask=valid)
                    plsc.addupdate_scatter(acc_v, [ab + 5], s2, mask=valid)
                    plsc.addupdate_scatter(acc_v, [ab + 6], vz, mask=valid)
                return cc
            lax.fori_loop(0, JROW // 4, _grp, 0)
            return carry
        lax.fori_loop(0, 8, _row, 0)
        return 0
    lax.fori_loop(u0, u1, _unit, 0)

    def _fold(t, _):
        v = acc_v[pl.ds(t * 16, 16)]
        for l in range(1, 16):
            v = v + acc_v[pl.ds(l * ACCW + t * 16, 16)]
        acc_v[pl.ds(t * 16, 16)] = v
        return 0
    lax.fori_loop(0, ACCW // 16, _fold, 0)
    pltpu.sync_copy(acc_v.at[pl.ds(0, ACCW)],
                    part_hbm.at[pl.ds(wid * ACCW, ACCW)])


def _vote_body(labe_hbm, dxm_hbm, dym_hbm, part_hbm,
               votes_hbm, sums_hbm,
               le_v, dx_v, dy_v, part_v, cx_v, cy_v, vacc_v):
    cid = lax.axis_index("c")
    sid = lax.axis_index("s")
    wid = cid * NS + sid
    b = wid // NS
    u0 = (wid * NUNIT) // NW
    u1 = ((wid + 1) * NUNIT) // NW
    lanes = lax.iota(jnp.int32, 16)

    pltpu.sync_copy(part_hbm, part_v)

    def _red(t, _):
        v = part_v[pl.ds(t * 16, 16)]
        for wk in range(1, NW):
            v = v + part_v[pl.ds(wk * ACCW + t * 16, 16)]
        part_v[pl.ds(t * 16, 16)] = v
        return 0
    lax.fori_loop(0, ACCW // 16, _red, 0)

    @pl.when(wid == 0)
    def _():
        pltpu.sync_copy(part_v.at[pl.ds(0, ACCW)], sums_hbm)

    def _slv(t, _):
        binv = t * 16 + lanes
        bi8 = binv * 8
        a11 = plsc.load_gather(part_v, [bi8 + 1])
        a12 = plsc.load_gather(part_v, [bi8 + 2])
        a22 = plsc.load_gather(part_v, [bi8 + 3])
        b1 = plsc.load_gather(part_v, [bi8 + 4])
        b2 = plsc.load_gather(part_v, [bi8 + 5])
        det = a11 * a22 - a12 * a12
        det = jnp.where(jnp.abs(det) < EPS, jnp.full((16,), EPS, F32), det)
        cx_v[pl.ds(t * 16, 16)] = (a22 * b1 - a12 * b2) / det
        cy_v[pl.ds(t * 16, 16)] = (a11 * b2 - a12 * b1) / det
        return 0
    lax.fori_loop(0, 3, _slv, 0)

    def _zero(i, _):
        vacc_v[pl.ds(i * 16, 16)] = jnp.zeros((16,), F32)
        return 0
    lax.fori_loop(0, (16 * VOTESTRIDE) // 16, _zero, 0)

    def _unit(u, _):
        r0 = u * 8
        pltpu.sync_copy(labe_hbm.at[pl.ds(r0, 8)], le_v)
        pltpu.sync_copy(dxm_hbm.at[pl.ds(r0, 8)], dx_v)
        pltpu.sync_copy(dym_hbm.at[pl.ds(r0, 8)], dy_v)

        def _row(rr, carry):
            yf = (r0 + rr - b * H).astype(F32)

            def _grp(j4, cc):
                for k in range(4):
                    cs = j4 * 64 + k * 16
                    le = le_v[rr, pl.ds(cs, 16)]
                    dxm = dx_v[rr, pl.ds(cs, 16)]
                    dym = dy_v[rr, pl.ds(cs, 16)]
                    binv = b * (NCLS - 1) + jnp.maximum(le - 1, 0)
                    cx = plsc.load_gather(cx_v, [binv])
                    cy = plsc.load_gather(cy_v, [binv])
                    xf = (cs + lanes).astype(F32)
                    ux = cx - xf
                    uy = cy - yf
                    u2 = jnp.maximum(ux * ux + uy * uy, 1e-12)
                    r2 = _rsqrt(u2)
                    num = ux * dxm + uy * dym
                    vm = num > 0.9 * (u2 * r2 + EPS)
                    one = jnp.ones((16,), F32)
                    plsc.addupdate_scatter(vacc_v,
                                           [lanes * VOTESTRIDE + binv],
                                           one, mask=vm)
                return cc
            lax.fori_loop(0, JROW // 4, _grp, 0)
            return carry
        lax.fori_loop(0, 8, _row, 0)
        return 0
    lax.fori_loop(u0, u1, _unit, 0)

    def _foldv(t, _):
        v = vacc_v[pl.ds(t * 16, 16)]
        for l in range(1, 16):
            v = v + vacc_v[pl.ds(l * VOTESTRIDE + t * 16, 16)]
        vacc_v[pl.ds(t * 16, 16)] = v
        return 0
    lax.fori_loop(0, VOTESTRIDE // 16, _foldv, 0)
    pltpu.sync_copy(vacc_v.at[pl.ds(0, VOTESTRIDE)],
                    votes_hbm.at[pl.ds(wid * VOTESTRIDE, VOTESTRIDE)])


def _fin_body(sums_ref, votes_ref, ext_ref, poses_ref, meta_ref,
              box_ref, pose_ref):
    sums = sums_ref[...]
    votes = jnp.sum(votes_ref[...], axis=0)[:NBIN]
    cnt = sums[:, 0]
    a11 = sums[:, 1]
    a12 = sums[:, 2]
    a22 = sums[:, 3]
    b1 = sums[:, 4]
    b2 = sums[:, 5]
    sz = sums[:, 6]
    det = a11 * a22 - a12 * a12
    det = jnp.where(jnp.abs(det) < EPS, EPS, det)
    cx = (a22 * b1 - a12 * b2) / det
    cy = (a11 * b2 - a12 * b1) / det

    bsel = lax.broadcasted_iota(jnp.int32, (NBIN,), 0) >= (NCLS - 1)
    fx = jnp.where(bsel, meta_ref[1, 0], meta_ref[0, 0]) + EPS
    fy = jnp.where(bsel, meta_ref[1, 4], meta_ref[0, 4]) + EPS
    px0 = jnp.where(bsel, meta_ref[1, 2], meta_ref[0, 2])
    py0 = jnp.where(bsel, meta_ref[1, 5], meta_ref[0, 5])

    frac = votes / jnp.maximum(cnt, 1.0)
    depth = jnp.exp(sz / jnp.maximum(cnt, 1.0))
    tx = depth * (cx - px0) / fx
    ty = depth * (cy - py0) / fy

    e = ext_ref[...]
    ext2 = jnp.sum(e * e, axis=1)
    e21 = ext2[1:NCLS]
    ext2sel = jnp.concatenate([e21, e21])
    extv = jnp.sqrt(ext2sel + EPS)
    bw = 0.5 * fx * extv / (depth + EPS)
    bh = 0.5 * fy * extv / (depth + EPS)
    validv = ((cnt > 500.0) & (votes >= 100.0)).astype(F32)
    score = frac * validv

    rowi = lax.broadcasted_iota(jnp.int32, (NBIN,), 0)
    clsv = (rowi + 1 - (NCLS - 1) * bsel.astype(jnp.int32)).astype(F32)
    box_ref[0, :] = bsel.astype(F32)
    box_ref[1, :] = clsv
    box_ref[2, :] = cx - bw
    box_ref[3, :] = cy - bh
    box_ref[4, :] = cx + bw
    box_ref[5, :] = cy + bh
    box_ref[6, :] = score

    for k in range(4):
        pose_ref[k, :] = jnp.where(bsel, poses_ref[1, 6 + k],
                                   poses_ref[0, 6 + k])
    pose_ref[4, :] = tx
    pose_ref[5, :] = ty
    pose_ref[6, :] = depth


_mesh = plsc.VectorSubcoreMesh(core_axis_name="c", subcore_axis_name="s")

_moments = pl.kernel(
    _mom_body,
    out_type=[jax.ShapeDtypeStruct((NW * ACCW,), F32)],
    mesh=_mesh,
    compiler_params=pltpu.CompilerParams(needs_layout_passes=False),
    scratch_types=[
        pltpu.VMEM((8, W), jnp.int32),
        pltpu.VMEM((8, W), F32),
        pltpu.VMEM((8, W), F32),
        pltpu.VMEM((8, W), F32),
        pltpu.VMEM((16 * ACCW,), F32),
    ],
)

_votes = pl.kernel(
    _vote_body,
    out_type=[
        jax.ShapeDtypeStruct((NW * VOTESTRIDE,), F32),
        jax.ShapeDtypeStruct((ACCW,), F32),
    ],
    mesh=_mesh,
    compiler_params=pltpu.CompilerParams(needs_layout_passes=False),
    scratch_types=[
        pltpu.VMEM((8, W), jnp.int32),
        pltpu.VMEM((8, W), F32),
        pltpu.VMEM((8, W), F32),
        pltpu.VMEM((NW * ACCW,), F32),
        pltpu.VMEM((VOTESTRIDE,), F32),
        pltpu.VMEM((VOTESTRIDE,), F32),
        pltpu.VMEM((16 * VOTESTRIDE,), F32),
    ],
)

_final = pl.pallas_call(
    _fin_body,
    out_shape=[
        jax.ShapeDtypeStruct((7, NBIN), F32),
        jax.ShapeDtypeStruct((7, NBIN), F32),
    ],
)


def kernel(labels, masks, vertex_pred, extents, poses, meta_data):
    lab2 = labels.reshape(RH, W)
    msk2 = masks.reshape(RH, W)
    labe, dxm, dym, vzm = _select(lab2, msk2, vertex_pred)
    part, = _moments(labe, dxm, dym, vzm)
    votes, sums = _votes(labe, dxm, dym, part)
    boxt, poset = _final(sums.reshape(NBIN, 8), votes.reshape(NW, VOTESTRIDE),
                         extents, poses, meta_data)
    top_box = boxt.T
    top_pose = poset.T
    top_target = jnp.zeros((NBIN, 4 * NCLS), F32)
    top_weight = jnp.zeros((NBIN, 4 * NCLS), F32)
    top_domain = jnp.repeat(jnp.arange(B, dtype=F32), NCLS - 1)
    return top_box, top_pose, top_target, top_weight, top_domain


# speedup vs baseline: 2.2395x; 1.0587x over previous
import numpy as np
import jax
import jax.numpy as jnp
from jax import lax
from jax.experimental import pallas as pl
from jax.experimental.pallas import tpu as pltpu
from jax.experimental.pallas import tpu_sc as plsc

B, H, W, NCLS = 2, 480, 640, 22
HW = H * W
NPIX = B * HW
RH = B * H
NC, NS = 2, 16
NW = NC * NS
NUNIT = RH // 8
JROW = W // 16
NBIN = B * (NCLS - 1)
ACCW = NBIN * 8
ACCL = ACCW + 1
VOTESTRIDE = 48
VOTEL = VOTESTRIDE + 1
BH = 16
EPS = 1e-6
F32 = jnp.float32


def _rsqrt(n2):
    i = lax.bitcast_convert_type(n2, jnp.int32)
    i = 0x5F3759DF - (i >> 1)
    r = lax.bitcast_convert_type(i, F32)
    for _ in range(3):
        r = r * (1.5 - 0.5 * n2 * r * r)
    return r


def _sel_body(lab_ref, msk_ref, vp_ref, labe_ref, dxm_ref, dym_ref, vzm_ref):
    lab = lab_ref[...]
    msk = msk_ref[...]
    le = jnp.where((msk > 0) & (lab > 0), lab, 0)
    vx = vp_ref[0, 0]
    vy = vp_ref[0, 1]
    vz = vp_ref[0, 2]
    for c in range(1, NCLS):
        mc = le == c
        vx = jnp.where(mc, vp_ref[0, 3 * c], vx)
        vy = jnp.where(mc, vp_ref[0, 3 * c + 1], vy)
        vz = jnp.where(mc, vp_ref[0, 3 * c + 2], vz)
    valid = le > 0
    n2 = jnp.maximum(vx * vx + vy * vy, 1e-12)
    r = lax.rsqrt(n2)
    s = r / (1.0 + EPS * r)
    zero = jnp.zeros_like(vx)
    labe_ref[...] = le
    dxm_ref[...] = jnp.where(valid, vx * s, zero)
    dym_ref[...] = jnp.where(valid, vy * s, zero)
    vzm_ref[...] = jnp.where(valid, vz, zero)


_plane_spec = pl.BlockSpec((BH, W), lambda g: (g, 0))

_select = pl.pallas_call(
    _sel_body,
    grid=(RH // BH,),
    in_specs=[
        _plane_spec,
        _plane_spec,
        pl.BlockSpec((1, 3 * NCLS, BH, W),
                     lambda g: (g // (H // BH), 0, g % (H // BH), 0)),
    ],
    out_specs=[_plane_spec, _plane_spec, _plane_spec, _plane_spec],
    out_shape=[
        jax.ShapeDtypeStruct((RH, W), jnp.int32),
        jax.ShapeDtypeStruct((RH, W), F32),
        jax.ShapeDtypeStruct((RH, W), F32),
        jax.ShapeDtypeStruct((RH, W), F32),
    ],
)


def _mom_body(labe_hbm, dxm_hbm, dym_hbm, vzm_hbm, part_hbm,
              le_v, dx_v, dy_v, vz_v, acc_v):
    cid = lax.axis_index("c")
    sid = lax.axis_index("s")
    wid = cid * NS + sid
    b = wid // NS
    u0 = (wid * NUNIT) // NW
    u1 = ((wid + 1) * NUNIT) // NW
    lanes = lax.iota(jnp.int32, 16)

    def _zero(i, _):
        acc_v[pl.ds(i * 16, 16)] = jnp.zeros((16,), F32)
        return 0
    lax.fori_loop(0, (16 * ACCL + 15) // 16, _zero, 0)

    def _unit(u, _):
        r0 = u * 8
        pltpu.sync_copy(labe_hbm.at[pl.ds(r0, 8)], le_v)
        pltpu.sync_copy(dxm_hbm.at[pl.ds(r0, 8)], dx_v)
        pltpu.sync_copy(dym_hbm.at[pl.ds(r0, 8)], dy_v)
        pltpu.sync_copy(vzm_hbm.at[pl.ds(r0, 8)], vz_v)

        def _row(rr, carry):
            yf = (r0 + rr - b * H).astype(F32)

            def _grp(j4, cc):
                for k in range(4):
                    cs = j4 * 64 + k * 16
                    le = le_v[rr, pl.ds(cs, 16)]
                    dx = dx_v[rr, pl.ds(cs, 16)]
                    dy = dy_v[rr, pl.ds(cs, 16)]
                    vz = vz_v[rr, pl.ds(cs, 16)]
                    valid = le > 0
                    xf = (cs + lanes).astype(F32)
                    t1 = 1.0 - dx * dx
                    t2 = -dx * dy
                    t3 = 1.0 - dy * dy
                    s1 = t1 * xf + t2 * yf
                    s2 = t2 * xf + t3 * yf
                    binv = b * (NCLS - 1) + jnp.maximum(le - 1, 0)
                    ab = lanes * ACCL + binv * 8
                    one = jnp.ones((16,), F32)
                    plsc.addupdate_scatter(acc_v, [ab], one, mask=valid)
                    plsc.addupdate_scatter(acc_v, [ab + 1], t1, mask=valid)
                    plsc.addupdate_scatter(acc_v, [ab + 2], t2, mask=valid)
                    plsc.addupdate_scatter(acc_v, [ab + 3], t3, mask=valid)
                    plsc.addupdate_scatter(acc_v, [ab + 4], s1, mask=valid)
                    plsc.addupdate_scatter(acc_v, [ab + 5], s2, mask=valid)
                    plsc.addupdate_scatter(acc_v, [ab + 6], vz, mask=valid)
                return cc
            lax.fori_loop(0, JROW // 4, _grp, 0)
            return carry
        lax.fori_loop(0, 8, _row, 0)
        return 0
    lax.fori_loop(u0, u1, _unit, 0)

    def _fold(t, _):
        v = acc_v[pl.ds(t * 16, 16)]
        for l in range(1, 16):
            v = v + acc_v[pl.ds(l * ACCL + t * 16, 16)]
        acc_v[pl.ds(t * 16, 16)] = v
        return 0
    lax.fori_loop(0, ACCW // 16, _fold, 0)
    pltpu.sync_copy(acc_v.at[pl.ds(0, ACCW)],
                    part_hbm.at[pl.ds(wid * ACCW, ACCW)])


def _vote_body(labe_hbm, dxm_hbm, dym_hbm, part_hbm,
               votes_hbm, sums_hbm,
               le_v, dx_v, dy_v, part_v, cx_v, cy_v, vacc_v):
    cid = lax.axis_index("c")
    sid = lax.axis_index("s")
    wid = cid * NS + sid
    b = wid // NS
    u0 = (wid * NUNIT) // NW
    u1 = ((wid + 1) * NUNIT) // NW
    lanes = lax.iota(jnp.int32, 16)

    pltpu.sync_copy(part_hbm, part_v)

    def _red(t, _):
        v = part_v[pl.ds(t * 16, 16)]
        for wk in range(1, NW):
            v = v + part_v[pl.ds(wk * ACCW + t * 16, 16)]
        part_v[pl.ds(t * 16, 16)] = v
        return 0
    lax.fori_loop(0, ACCW // 16, _red, 0)

    @pl.when(wid == 0)
    def _():
        pltpu.sync_copy(part_v.at[pl.ds(0, ACCW)], sums_hbm)

    def _slv(t, _):
        binv = t * 16 + lanes
        bi8 = binv * 8
        a11 = plsc.load_gather(part_v, [bi8 + 1])
        a12 = plsc.load_gather(part_v, [bi8 + 2])
        a22 = plsc.load_gather(part_v, [bi8 + 3])
        b1 = plsc.load_gather(part_v, [bi8 + 4])
        b2 = plsc.load_gather(part_v, [bi8 + 5])
        det = a11 * a22 - a12 * a12
        det = jnp.where(jnp.abs(det) < EPS, jnp.full((16,), EPS, F32), det)
        cx_v[pl.ds(t * 16, 16)] = (a22 * b1 - a12 * b2) / det
        cy_v[pl.ds(t * 16, 16)] = (a11 * b2 - a12 * b1) / det
        return 0
    lax.fori_loop(0, 3, _slv, 0)

    def _zero(i, _):
        vacc_v[pl.ds(i * 16, 16)] = jnp.zeros((16,), F32)
        return 0
    lax.fori_loop(0, (16 * VOTEL + 15) // 16, _zero, 0)

    def _unit(u, _):
        r0 = u * 8
        pltpu.sync_copy(labe_hbm.at[pl.ds(r0, 8)], le_v)
        pltpu.sync_copy(dxm_hbm.at[pl.ds(r0, 8)], dx_v)
        pltpu.sync_copy(dym_hbm.at[pl.ds(r0, 8)], dy_v)

        def _row(rr, carry):
            yf = (r0 + rr - b * H).astype(F32)

            def _grp(j4, cc):
                for k in range(4):
                    cs = j4 * 64 + k * 16
                    le = le_v[rr, pl.ds(cs, 16)]
                    dxm = dx_v[rr, pl.ds(cs, 16)]
                    dym = dy_v[rr, pl.ds(cs, 16)]
                    binv = b * (NCLS - 1) + jnp.maximum(le - 1, 0)
                    cx = plsc.load_gather(cx_v, [binv])
                    cy = plsc.load_gather(cy_v, [binv])
                    xf = (cs + lanes).astype(F32)
                    ux = cx - xf
                    uy = cy - yf
                    u2 = jnp.maximum(ux * ux + uy * uy, 1e-12)
                    r2 = _rsqrt(u2)
                    num = ux * dxm + uy * dym
                    vm = num > 0.9 * (u2 * r2 + EPS)
                    one = jnp.ones((16,), F32)
                    plsc.addupdate_scatter(vacc_v,
                                           [lanes * VOTEL + binv],
                                           one, mask=vm)
                return cc
            lax.fori_loop(0, JROW // 4, _grp, 0)
            return carry
        lax.fori_loop(0, 8, _row, 0)
        return 0
    lax.fori_loop(u0, u1, _unit, 0)

    def _foldv(t, _):
        v = vacc_v[pl.ds(t * 16, 16)]
        for l in range(1, 16):
            v = v + vacc_v[pl.ds(l * VOTEL + t * 16, 16)]
        vacc_v[pl.ds(t * 16, 16)] = v
        return 0
    lax.fori_loop(0, VOTESTRIDE // 16, _foldv, 0)
    pltpu.sync_copy(vacc_v.at[pl.ds(0, VOTESTRIDE)],
                    votes_hbm.at[pl.ds(wid * VOTESTRIDE, VOTESTRIDE)])


def _fin_body(sums_ref, votes_ref, ext_ref, poses_ref, meta_ref,
              box_ref, pose_ref):
    sums = sums_ref[...]
    votes = jnp.sum(votes_ref[...], axis=0)[:NBIN]
    cnt = sums[:, 0]
    a11 = sums[:, 1]
    a12 = sums[:, 2]
    a22 = sums[:, 3]
    b1 = sums[:, 4]
    b2 = sums[:, 5]
    sz = sums[:, 6]
    det = a11 * a22 - a12 * a12
    det = jnp.where(jnp.abs(det) < EPS, EPS, det)
    cx = (a22 * b1 - a12 * b2) / det
    cy = (a11 * b2 - a12 * b1) / det

    bsel = lax.broadcasted_iota(jnp.int32, (NBIN,), 0) >= (NCLS - 1)
    fx = jnp.where(bsel, meta_ref[1, 0], meta_ref[0, 0]) + EPS
    fy = jnp.where(bsel, meta_ref[1, 4], meta_ref[0, 4]) + EPS
    px0 = jnp.where(bsel, meta_ref[1, 2], meta_ref[0, 2])
    py0 = jnp.where(bsel, meta_ref[1, 5], meta_ref[0, 5])

    frac = votes / jnp.maximum(cnt, 1.0)
    depth = jnp.exp(sz / jnp.maximum(cnt, 1.0))
    tx = depth * (cx - px0) / fx
    ty = depth * (cy - py0) / fy

    e = ext_ref[...]
    ext2 = jnp.sum(e * e, axis=1)
    e21 = ext2[1:NCLS]
    ext2sel = jnp.concatenate([e21, e21])
    extv = jnp.sqrt(ext2sel + EPS)
    bw = 0.5 * fx * extv / (depth + EPS)
    bh = 0.5 * fy * extv / (depth + EPS)
    validv = ((cnt > 500.0) & (votes >= 100.0)).astype(F32)
    score = frac * validv

    rowi = lax.broadcasted_iota(jnp.int32, (NBIN,), 0)
    clsv = (rowi + 1 - (NCLS - 1) * bsel.astype(jnp.int32)).astype(F32)
    box_ref[0, :] = bsel.astype(F32)
    box_ref[1, :] = clsv
    box_ref[2, :] = cx - bw
    box_ref[3, :] = cy - bh
    box_ref[4, :] = cx + bw
    box_ref[5, :] = cy + bh
    box_ref[6, :] = score

    for k in range(4):
        pose_ref[k, :] = jnp.where(bsel, poses_ref[1, 6 + k],
                                   poses_ref[0, 6 + k])
    pose_ref[4, :] = tx
    pose_ref[5, :] = ty
    pose_ref[6, :] = depth


_mesh = plsc.VectorSubcoreMesh(core_axis_name="c", subcore_axis_name="s")

_moments = pl.kernel(
    _mom_body,
    out_type=[jax.ShapeDtypeStruct((NW * ACCW,), F32)],
    mesh=_mesh,
    compiler_params=pltpu.CompilerParams(needs_layout_passes=False),
    scratch_types=[
        pltpu.VMEM((8, W), jnp.int32),
        pltpu.VMEM((8, W), F32),
        pltpu.VMEM((8, W), F32),
        pltpu.VMEM((8, W), F32),
        pltpu.VMEM((16 * ACCL + 16,), F32),
    ],
)

_votes = pl.kernel(
    _vote_body,
    out_type=[
        jax.ShapeDtypeStruct((NW * VOTESTRIDE,), F32),
        jax.ShapeDtypeStruct((ACCW,), F32),
    ],
    mesh=_mesh,
    compiler_params=pltpu.CompilerParams(needs_layout_passes=False),
    scratch_types=[
        pltpu.VMEM((8, W), jnp.int32),
        pltpu.VMEM((8, W), F32),
        pltpu.VMEM((8, W), F32),
        pltpu.VMEM((NW * ACCW,), F32),
        pltpu.VMEM((VOTESTRIDE,), F32),
        pltpu.VMEM((VOTESTRIDE,), F32),
        pltpu.VMEM((16 * VOTEL + 16,), F32),
    ],
)

_final = pl.pallas_call(
    _fin_body,
    out_shape=[
        jax.ShapeDtypeStruct((7, NBIN), F32),
        jax.ShapeDtypeStruct((7, NBIN), F32),
    ],
)


def kernel(labels, masks, vertex_pred, extents, poses, meta_data):
    lab2 = labels.reshape(RH, W)
    msk2 = masks.reshape(RH, W)
    labe, dxm, dym, vzm = _select(lab2, msk2, vertex_pred)
    part, = _moments(labe, dxm, dym, vzm)
    votes, sums = _votes(labe, dxm, dym, part)
    boxt, poset = _final(sums.reshape(NBIN, 8), votes.reshape(NW, VOTESTRIDE),
                         extents, poses, meta_data)
    top_box = boxt.T
    top_pose = poset.T
    top_target = jnp.zeros((NBIN, 4 * NCLS), F32)
    top_weight = jnp.zeros((NBIN, 4 * NCLS), F32)
    top_domain = jnp.repeat(jnp.arange(B, dtype=F32), NCLS - 1)
    return top_box, top_pose, top_target, top_weight, top_domain
